# Initial kernel scaffold; baseline (speedup 1.0000x reference)
import jax
import jax.numpy as jnp
from jax.experimental import pallas as pl


def kernel(x, edge_index, W1, b1, gamma1, beta1, W2, b2, gamma2, beta2):
    raise NotImplementedError("write your pallas kernel here")


# speedup vs baseline: 17.6335x; 17.6335x over previous
import functools

import jax
import jax.numpy as jnp
from jax import lax
from jax.experimental import pallas as pl
from jax.experimental.pallas import tpu as pltpu
from jax.experimental.pallas import tpu_sc as plsc

N = 10000
E = 320000
NP = 10240
NC = 2
NS = 16
NW = NC * NS
EPW = E // NW
C = 80
ITERS = EPW // C
RPT = NP // NS
DEGW = 16

_MESH = plsc.VectorSubcoreMesh(core_axis_name="c", subcore_axis_name="s")
_SC_PARAMS = pltpu.CompilerParams(use_tc_tiling_on_sc=False)
_F32 = jnp.float32
_HIGHEST = lax.Precision.HIGHEST


@functools.partial(
    pl.kernel,
    out_type=jax.ShapeDtypeStruct((NC, NP, DEGW), _F32),
    mesh=_MESH,
    scratch_types=[
        pltpu.VMEM_SHARED((NP, DEGW), _F32),
        pltpu.VMEM((ITERS, C), jnp.int32),
        pltpu.VMEM((C, DEGW), _F32),
    ],
    compiler_params=_SC_PARAMS,
)
def _sc_degree(dst_hbm, zeros_hbm, e1_hbm, out_hbm, acc, dst_v, e1_v):
    cid = lax.axis_index("c")
    sid = lax.axis_index("s")
    wid = sid * NC + cid
    pltpu.sync_copy(zeros_hbm.at[pl.ds(sid * RPT, RPT)], acc.at[pl.ds(sid * RPT, RPT)])
    pltpu.sync_copy(dst_hbm.at[wid], dst_v)
    pltpu.sync_copy(e1_hbm, e1_v)
    plsc.subcore_barrier()

    def body(i, carry):
        pltpu.sync_copy(e1_v, acc.at[dst_v.at[i]], add=True)
        return carry

    lax.fori_loop(0, ITERS, body, 0)
    plsc.subcore_barrier()
    pltpu.sync_copy(acc.at[pl.ds(sid * RPT, RPT)],
                    out_hbm.at[cid, pl.ds(sid * RPT, RPT)])


def _make_sc_agg(d):
    @functools.partial(
        pl.kernel,
        out_type=jax.ShapeDtypeStruct((NC, NP, d), _F32),
        mesh=_MESH,
        scratch_types=[
            pltpu.VMEM_SHARED((NP, d), _F32),
            pltpu.VMEM((ITERS, C), jnp.int32),
            pltpu.VMEM((ITERS, C), jnp.int32),
            pltpu.VMEM((C, d), _F32),
            pltpu.SemaphoreType.DMA,
        ],
        compiler_params=_SC_PARAMS,
    )
    def _sc_agg(h_hbm, src_hbm, dst_hbm, zeros_hbm, out_hbm,
                acc, src_v, dst_v, rows_v, sem):
        cid = lax.axis_index("c")
        sid = lax.axis_index("s")
        wid = sid * NC + cid
        pltpu.sync_copy(zeros_hbm.at[pl.ds(sid * RPT, RPT)],
                        acc.at[pl.ds(sid * RPT, RPT)])
        pltpu.sync_copy(src_hbm.at[wid], src_v)
        pltpu.sync_copy(dst_hbm.at[wid], dst_v)
        plsc.subcore_barrier()

        def body(i, carry):
            pltpu.async_copy(h_hbm.at[src_v.at[i]], rows_v, sem).wait()
            pltpu.sync_copy(rows_v, acc.at[dst_v.at[i]], add=True)
            return carry

        lax.fori_loop(0, ITERS, body, 0)
        plsc.subcore_barrier()
        pltpu.sync_copy(acc.at[pl.ds(sid * RPT, RPT)],
                        out_hbm.at[cid, pl.ds(sid * RPT, RPT)])

    return _sc_agg


_sc_agg1 = _make_sc_agg(128)
_sc_agg2 = _make_sc_agg(64)


def _tc_a(deg_ref, x_ref, w1_ref, h_ref, dis_ref):
    deg = deg_ref[0, :, 0:1] + deg_ref[1, :, 0:1]
    dis = jnp.where(deg > 0, lax.rsqrt(jnp.maximum(deg, 1e-12)), 0.0)
    dis_ref[...] = dis
    h = jnp.dot(x_ref[...], w1_ref[...], preferred_element_type=_F32,
                precision=_HIGHEST)
    h_ref[0:N, :] = h * dis[0:N]
    h_ref[N:NP, :] = jnp.zeros((NP - N, h.shape[1]), _F32)


def _tc_b(agg_ref, dis_ref, b1_ref, g1_ref, be1_ref, w2_ref, h2_ref):
    dis = dis_ref[0:N]
    out1 = (agg_ref[0, 0:N, :] + agg_ref[1, 0:N, :]) * dis + b1_ref[...]
    mean = jnp.mean(out1, axis=0, keepdims=True)
    var = jnp.mean((out1 - mean) ** 2, axis=0, keepdims=True)
    h = (out1 - mean) * lax.rsqrt(var + 1e-5) * g1_ref[...] + be1_ref[...]
    h = jnp.maximum(h, 0.0)
    h2 = jnp.dot(h, w2_ref[...], preferred_element_type=_F32,
                 precision=_HIGHEST)
    h2_ref[0:N, :] = h2 * dis
    h2_ref[N:NP, :] = jnp.zeros((NP - N, h2.shape[1]), _F32)


def _tc_c(agg_ref, dis_ref, b2_ref, g2_ref, be2_ref, out_ref):
    dis = dis_ref[0:N]
    out2 = (agg_ref[0, 0:N, :] + agg_ref[1, 0:N, :]) * dis + b2_ref[...]
    mean = jnp.mean(out2, axis=0, keepdims=True)
    var = jnp.mean((out2 - mean) ** 2, axis=0, keepdims=True)
    h = (out2 - mean) * lax.rsqrt(var + 1e-5) * g2_ref[...] + be2_ref[...]
    out_ref[...] = jnp.maximum(h, 0.0)


def kernel(x, edge_index, W1, b1, gamma1, beta1, W2, b2, gamma2, beta2):
    src = edge_index[0].astype(jnp.int32).reshape(NW, ITERS, C)
    dst = edge_index[1].astype(jnp.int32).reshape(NW, ITERS, C)
    zeros128 = jnp.zeros((NP, 128), _F32)
    zeros64 = jnp.zeros((NP, 64), _F32)
    zerosw = jnp.zeros((NP, DEGW), _F32)
    e1 = jnp.zeros((C, DEGW), _F32).at[:, 0].set(1.0)

    deg_p = _sc_degree(dst, zerosw, e1)

    h1s, dis = pl.pallas_call(
        _tc_a,
        out_shape=[jax.ShapeDtypeStruct((NP, 128), _F32),
                   jax.ShapeDtypeStruct((NP, 1), _F32)],
    )(deg_p, x, W1)

    agg1 = _sc_agg1(h1s, src, dst, zeros128)

    h2s = pl.pallas_call(
        _tc_b,
        out_shape=jax.ShapeDtypeStruct((NP, 64), _F32),
    )(agg1, dis, b1.reshape(1, 128), gamma1.reshape(1, 128),
      beta1.reshape(1, 128), W2)

    agg2 = _sc_agg2(h2s, src, dst, zeros64)

    out = pl.pallas_call(
        _tc_c,
        out_shape=jax.ShapeDtypeStruct((N, 64), _F32),
    )(agg2, dis, b2.reshape(1, 64), gamma2.reshape(1, 64),
      beta2.reshape(1, 64))
    return out
